# Initial kernel scaffold; baseline (speedup 1.0000x reference)
import jax
import jax.numpy as jnp
from jax.experimental import pallas as pl


def kernel(features, edge_index, W1, b1, W2, b2, W3, b3):
    raise NotImplementedError("write your pallas kernel here")


# speedup vs baseline: 4.0970x; 4.0970x over previous
import functools

import jax
import jax.numpy as jnp
from jax import lax
from jax.experimental import pallas as pl
from jax.experimental.pallas import tpu as pltpu
from jax.experimental.pallas import tpu_sc as plsc

N = 10000
E = 320000
D = 128
NC = 2
NS = 16
NW = NC * NS
EPW = E // NW
CHUNK = 80
NCHUNKS = EPW // CHUNK
NRCHUNKS = N // CHUNK
CL = 16
RBLK = 1000
assert EPW % CHUNK == 0 and N % CHUNK == 0 and N % RBLK == 0
assert (EPW % 8 == 0) and (CHUNK % 8 == 0)


def _strided_row_chunks(s, body):
    def it(i, _):
        j = s + i * NS

        @pl.when(j < NRCHUNKS)
        def _():
            body(j * CHUNK)
        return 0

    lax.fori_loop(0, (NRCHUNKS + NS - 1) // NS, it, 0)


def _mesh():
    return plsc.VectorSubcoreMesh(core_axis_name="c", subcore_axis_name="s")


SRC_LANE = 0
DST_LANE = 64


def _count_body(src_hbm, dst_hbm, out_c, acc, buf_s, buf_d, idx_s, idx_d, sem):
    c = lax.axis_index("c")
    s = lax.axis_index("s")
    wid = c * NS + s

    marker = jnp.where(lax.iota(jnp.int32, 16) == 0, 1.0, 0.0)
    zeros16 = jnp.zeros((16,), jnp.float32)

    def zero_bufs(i, _):
        for j in range(D // 16):
            buf_s[i, pl.ds(j * 16, 16)] = zeros16
            buf_d[i, pl.ds(j * 16, 16)] = zeros16
        return 0

    lax.fori_loop(0, CHUNK, zero_bufs, 0)
    _strided_row_chunks(s, lambda rbase: pltpu.sync_copy(
        buf_s, acc.at[pl.ds(rbase, CHUNK)]))

    def mark_bufs(i, _):
        buf_s[i, pl.ds(SRC_LANE, 16)] = marker
        buf_d[i, pl.ds(DST_LANE, 16)] = marker
        return 0

    lax.fori_loop(0, CHUNK, mark_bufs, 0)
    plsc.subcore_barrier()

    def chunk_body(k, _):
        ebase = wid * EPW + k * CHUNK
        pltpu.sync_copy(src_hbm.at[pl.ds(ebase, CHUNK)], idx_s)
        pltpu.sync_copy(dst_hbm.at[pl.ds(ebase, CHUNK)], idx_d)
        pltpu.sync_copy(buf_s, acc.at[idx_s], add=True)
        pltpu.sync_copy(buf_d, acc.at[idx_d], add=True)
        return 0

    lax.fori_loop(0, NCHUNKS, chunk_body, 0)
    plsc.subcore_barrier()
    _strided_row_chunks(s, lambda rbase: pltpu.sync_copy(
        acc.at[pl.ds(rbase, CHUNK)], out_c.at[c, pl.ds(rbase, CHUNK)]))


def _sc_degree_counts(src, dst):
    kern = pl.kernel(
        _count_body,
        out_type=jax.ShapeDtypeStruct((NC, N, D), jnp.float32),
        mesh=_mesh(),
        scratch_types=[
            pltpu.VMEM_SHARED((N, D), jnp.float32),
            pltpu.VMEM((CHUNK, D), jnp.float32),
            pltpu.VMEM((CHUNK, D), jnp.float32),
            pltpu.VMEM((CHUNK,), jnp.int32),
            pltpu.VMEM((CHUNK,), jnp.int32),
            pltpu.SemaphoreType.DMA,
        ],
    )
    return kern(src, dst)


def _scatter_body(y_hbm, src_hbm, dst_hbm, out_hbm,
                  acc, idx_s, idx_d, rows, sem):
    c = lax.axis_index("c")
    s = lax.axis_index("s")
    wid = c * NS + s

    def zrow(i, _):
        for j in range(D // 16):
            rows[i, pl.ds(j * 16, 16)] = jnp.zeros((16,), jnp.float32)
        return 0

    lax.fori_loop(0, CHUNK, zrow, 0)
    _strided_row_chunks(s, lambda rbase: pltpu.sync_copy(
        rows, acc.at[pl.ds(rbase, CHUNK)]))
    plsc.subcore_barrier()

    def chunk_body(k, _):
        ebase = wid * EPW + k * CHUNK
        pltpu.sync_copy(src_hbm.at[pl.ds(ebase, CHUNK)], idx_s)
        pltpu.sync_copy(dst_hbm.at[pl.ds(ebase, CHUNK)], idx_d)
        pltpu.async_copy(y_hbm.at[idx_s], rows, sem).wait()
        pltpu.sync_copy(rows, acc.at[idx_d], add=True)
        return 0

    lax.fori_loop(0, NCHUNKS, chunk_body, 0)
    plsc.subcore_barrier()
    _strided_row_chunks(s, lambda rbase: pltpu.sync_copy(
        acc.at[pl.ds(rbase, CHUNK)], out_hbm.at[c, pl.ds(rbase, CHUNK)]))


def _sc_scatter(y, src, dst):
    kern = pl.kernel(
        _scatter_body,
        out_type=jax.ShapeDtypeStruct((NC, N, D), jnp.float32),
        mesh=_mesh(),
        scratch_types=[
            pltpu.VMEM_SHARED((N, D), jnp.float32),
            pltpu.VMEM((CHUNK,), jnp.int32),
            pltpu.VMEM((CHUNK,), jnp.int32),
            pltpu.VMEM((CHUNK, D), jnp.float32),
            pltpu.SemaphoreType.DMA,
        ],
    )
    return kern(y, src, dst)


def _norm_from_counts(cnt, lane):
    deg = cnt[0, :, lane:lane + 1] + cnt[1, :, lane:lane + 1]
    return jnp.where(deg > 0, lax.rsqrt(jnp.maximum(deg, 1.0)), 0.0)


def _dot(x, w):
    return jnp.dot(x, w, preferred_element_type=jnp.float32,
                   precision=lax.Precision.HIGHEST)


def _prep_body(x_ref, c_ref, w_ref, o_ref):
    cnt = c_ref[...]
    ns = _norm_from_counts(cnt, SRC_LANE)
    o_ref[...] = _dot(x_ref[...] * ns, w_ref[...])


def _mid_body(z_ref, c_ref, b_ref, w_ref, o_ref):
    cnt = c_ref[...]
    nd = _norm_from_counts(cnt, DST_LANE)
    ns = _norm_from_counts(cnt, SRC_LANE)
    x = jnp.maximum(nd * (z_ref[0] + z_ref[1]) + b_ref[...], 0.0)
    o_ref[...] = _dot(x * ns, w_ref[...])


def _final_body(z_ref, c_ref, b_ref, o_ref):
    nd = _norm_from_counts(c_ref[...], DST_LANE)
    v = nd * (z_ref[0] + z_ref[1]) + b_ref[...]
    t = v - jnp.max(v, axis=1, keepdims=True)
    o_ref[...] = t - jnp.log(jnp.sum(jnp.exp(t), axis=1, keepdims=True))


_Z_SPEC = pl.BlockSpec((NC, RBLK, D), lambda i: (0, i, 0))
_B_SPEC = pl.BlockSpec((1, D), lambda i: (0, 0))
_W_SPEC = pl.BlockSpec((D, D), lambda i: (0, 0))
_X_SPEC = pl.BlockSpec((RBLK, D), lambda i: (i, 0))
_OUT_SHAPE = jax.ShapeDtypeStruct((N, D), jnp.float32)


def _tc_prep(x, cnts, w):
    return pl.pallas_call(
        _prep_body, grid=(N // RBLK,),
        in_specs=[_X_SPEC, _Z_SPEC, _W_SPEC],
        out_specs=_X_SPEC, out_shape=_OUT_SHAPE,
    )(x, cnts, w)


def _tc_mid(z, cnts, b, w):
    return pl.pallas_call(
        _mid_body, grid=(N // RBLK,),
        in_specs=[_Z_SPEC, _Z_SPEC, _B_SPEC, _W_SPEC],
        out_specs=_X_SPEC, out_shape=_OUT_SHAPE,
    )(z, cnts, b, w)


def _tc_final(z, cnts, b):
    return pl.pallas_call(
        _final_body, grid=(N // RBLK,),
        in_specs=[_Z_SPEC, _Z_SPEC, _B_SPEC],
        out_specs=_X_SPEC, out_shape=_OUT_SHAPE,
    )(z, cnts, b)


def kernel(features, edge_index, W1, b1, W2, b2, W3, b3):
    src = edge_index[0]
    dst = edge_index[1]
    b1r = b1.reshape(1, D)
    b2r = b2.reshape(1, D)
    b3r = b3.reshape(1, D)
    cnts = _sc_degree_counts(src, dst)
    y1 = _tc_prep(features, cnts, W1)
    z1 = _sc_scatter(y1, src, dst)
    y2 = _tc_mid(z1, cnts, b1r, W2)
    z2 = _sc_scatter(y2, src, dst)
    y3 = _tc_mid(z2, cnts, b2r, W3)
    z3 = _sc_scatter(y3, src, dst)
    return _tc_final(z3, cnts, b3r)


# speedup vs baseline: 7.5260x; 1.8370x over previous
import functools

import jax
import jax.numpy as jnp
from jax import lax
from jax.experimental import pallas as pl
from jax.experimental.pallas import tpu as pltpu
from jax.experimental.pallas import tpu_sc as plsc

N = 10000
E = 320000
D = 128
NC = 2
NS = 16
NW = NC * NS
EPW = E // NW
CHUNK = 80
NCHUNKS = EPW // CHUNK
NBUF = 5
SBUF = 4
SFULL = NCHUNKS // SBUF
STAIL = NCHUNKS % SBUF
NRCHUNKS = N // CHUNK
CL = 16
RBLK = 1000
assert EPW % CHUNK == 0 and N % CHUNK == 0 and N % RBLK == 0
assert (EPW % 8 == 0) and (CHUNK % 8 == 0) and NCHUNKS % NBUF == 0


def _strided_row_chunks(s, body):
    def it(i, _):
        j = s + i * NS

        @pl.when(j < NRCHUNKS)
        def _():
            body(j * CHUNK)
        return 0

    lax.fori_loop(0, (NRCHUNKS + NS - 1) // NS, it, 0)


def _mesh():
    return plsc.VectorSubcoreMesh(core_axis_name="c", subcore_axis_name="s")


SRC_LANE = 0
DST_LANE = 64


def _count_body(src_hbm, dst_hbm, out_c, acc, buf_s, buf_d, idx_s, idx_d,
                isem, asem):
    c = lax.axis_index("c")
    s = lax.axis_index("s")
    wid = c * NS + s

    marker = jnp.where(lax.iota(jnp.int32, 16) == 0, 1.0, 0.0)
    zeros16 = jnp.zeros((16,), jnp.float32)

    def zero_bufs(i, _):
        for j in range(D // 16):
            buf_s[i, pl.ds(j * 16, 16)] = zeros16
            buf_d[i, pl.ds(j * 16, 16)] = zeros16
        return 0

    lax.fori_loop(0, CHUNK, zero_bufs, 0)
    _strided_row_chunks(s, lambda rbase: pltpu.sync_copy(
        buf_s, acc.at[pl.ds(rbase, CHUNK)]))

    def mark_bufs(i, _):
        buf_s[i, pl.ds(SRC_LANE, 16)] = marker
        buf_d[i, pl.ds(DST_LANE, 16)] = marker
        return 0

    lax.fori_loop(0, CHUNK, mark_bufs, 0)
    plsc.subcore_barrier()

    def group_body(g, _):
        j0 = g * NBUF
        il, al = [], []
        for b in range(NBUF):
            ebase = wid * EPW + (j0 + b) * CHUNK
            il.append((
                pltpu.async_copy(src_hbm.at[pl.ds(ebase, CHUNK)],
                                 idx_s.at[b], isem.at[b]),
                pltpu.async_copy(dst_hbm.at[pl.ds(ebase, CHUNK)],
                                 idx_d.at[b], isem.at[b]),
            ))
        for b in range(NBUF):
            il[b][0].wait()
            il[b][1].wait()
            al.append((
                pltpu.async_copy(buf_s, acc.at[idx_s.at[b]], asem.at[b],
                                 add=True),
                pltpu.async_copy(buf_d, acc.at[idx_d.at[b]], asem.at[b],
                                 add=True),
            ))
        for b in range(NBUF):
            al[b][0].wait()
            al[b][1].wait()
        return 0

    lax.fori_loop(0, NCHUNKS // NBUF, group_body, 0)
    plsc.subcore_barrier()
    _strided_row_chunks(s, lambda rbase: pltpu.sync_copy(
        acc.at[pl.ds(rbase, CHUNK)], out_c.at[c, pl.ds(rbase, CHUNK)]))


def _sc_degree_counts(src, dst):
    kern = pl.kernel(
        _count_body,
        out_type=jax.ShapeDtypeStruct((NC, N, D), jnp.float32),
        mesh=_mesh(),
        scratch_types=[
            pltpu.VMEM_SHARED((N, D), jnp.float32),
            pltpu.VMEM((CHUNK, D), jnp.float32),
            pltpu.VMEM((CHUNK, D), jnp.float32),
            pltpu.VMEM((NBUF, CHUNK), jnp.int32),
            pltpu.VMEM((NBUF, CHUNK), jnp.int32),
            pltpu.SemaphoreType.DMA((NBUF,)),
            pltpu.SemaphoreType.DMA((NBUF,)),
        ],
    )
    return kern(src, dst)


def _scatter_body(y_hbm, src_hbm, dst_hbm, out_hbm,
                  acc, idx_s, idx_d, rows, isem, gsem, ssem):
    c = lax.axis_index("c")
    s = lax.axis_index("s")
    wid = c * NS + s

    def zrow(i, _):
        for j in range(D // 16):
            rows[0, i, pl.ds(j * 16, 16)] = jnp.zeros((16,), jnp.float32)
        return 0

    lax.fori_loop(0, CHUNK, zrow, 0)
    _strided_row_chunks(s, lambda rbase: pltpu.sync_copy(
        rows.at[0], acc.at[pl.ds(rbase, CHUNK)]))
    plsc.subcore_barrier()

    def group_body(g, _):
        j0 = g * SBUF
        il, gl, sl = [], [], []
        for b in range(SBUF):
            ebase = wid * EPW + (j0 + b) * CHUNK
            il.append((
                pltpu.async_copy(src_hbm.at[pl.ds(ebase, CHUNK)],
                                 idx_s.at[b], isem.at[b]),
                pltpu.async_copy(dst_hbm.at[pl.ds(ebase, CHUNK)],
                                 idx_d.at[b], isem.at[b]),
            ))
        for b in range(SBUF):
            il[b][0].wait()
            gl.append(pltpu.async_copy(y_hbm.at[idx_s.at[b]], rows.at[b],
                                       gsem.at[b]))
        for b in range(SBUF):
            gl[b].wait()
            il[b][1].wait()
            sl.append(pltpu.async_copy(rows.at[b], acc.at[idx_d.at[b]],
                                       ssem.at[b], add=True))
        for b in range(SBUF):
            sl[b].wait()
        return 0

    lax.fori_loop(0, SFULL, group_body, 0)
    for t in range(STAIL):
        ebase = wid * EPW + (SFULL * SBUF + t) * CHUNK
        pltpu.sync_copy(src_hbm.at[pl.ds(ebase, CHUNK)], idx_s.at[0])
        pltpu.sync_copy(dst_hbm.at[pl.ds(ebase, CHUNK)], idx_d.at[0])
        pltpu.async_copy(y_hbm.at[idx_s.at[0]], rows.at[0], gsem.at[0]).wait()
        pltpu.sync_copy(rows.at[0], acc.at[idx_d.at[0]], add=True)
    plsc.subcore_barrier()
    _strided_row_chunks(s, lambda rbase: pltpu.sync_copy(
        acc.at[pl.ds(rbase, CHUNK)], out_hbm.at[c, pl.ds(rbase, CHUNK)]))


def _sc_scatter(y, src, dst):
    kern = pl.kernel(
        _scatter_body,
        out_type=jax.ShapeDtypeStruct((NC, N, D), jnp.float32),
        mesh=_mesh(),
        scratch_types=[
            pltpu.VMEM_SHARED((N, D), jnp.float32),
            pltpu.VMEM((SBUF, CHUNK), jnp.int32),
            pltpu.VMEM((SBUF, CHUNK), jnp.int32),
            pltpu.VMEM((SBUF, CHUNK, D), jnp.float32),
            pltpu.SemaphoreType.DMA((SBUF,)),
            pltpu.SemaphoreType.DMA((SBUF,)),
            pltpu.SemaphoreType.DMA((SBUF,)),
        ],
    )
    return kern(y, src, dst)


def _norm_from_counts(cnt, lane):
    deg = cnt[0, :, lane:lane + 1] + cnt[1, :, lane:lane + 1]
    return jnp.where(deg > 0, lax.rsqrt(jnp.maximum(deg, 1.0)), 0.0)


def _dot(x, w):
    return jnp.dot(x, w, preferred_element_type=jnp.float32,
                   precision=lax.Precision.HIGHEST)


def _prep_body(x_ref, c_ref, w_ref, o_ref):
    cnt = c_ref[...]
    ns = _norm_from_counts(cnt, SRC_LANE)
    o_ref[...] = _dot(x_ref[...] * ns, w_ref[...])


def _mid_body(z_ref, c_ref, b_ref, w_ref, o_ref):
    cnt = c_ref[...]
    nd = _norm_from_counts(cnt, DST_LANE)
    ns = _norm_from_counts(cnt, SRC_LANE)
    x = jnp.maximum(nd * (z_ref[0] + z_ref[1]) + b_ref[...], 0.0)
    o_ref[...] = _dot(x * ns, w_ref[...])


def _final_body(z_ref, c_ref, b_ref, o_ref):
    nd = _norm_from_counts(c_ref[...], DST_LANE)
    v = nd * (z_ref[0] + z_ref[1]) + b_ref[...]
    t = v - jnp.max(v, axis=1, keepdims=True)
    o_ref[...] = t - jnp.log(jnp.sum(jnp.exp(t), axis=1, keepdims=True))


_Z_SPEC = pl.BlockSpec((NC, RBLK, D), lambda i: (0, i, 0))
_B_SPEC = pl.BlockSpec((1, D), lambda i: (0, 0))
_W_SPEC = pl.BlockSpec((D, D), lambda i: (0, 0))
_X_SPEC = pl.BlockSpec((RBLK, D), lambda i: (i, 0))
_OUT_SHAPE = jax.ShapeDtypeStruct((N, D), jnp.float32)


def _tc_prep(x, cnts, w):
    return pl.pallas_call(
        _prep_body, grid=(N // RBLK,),
        in_specs=[_X_SPEC, _Z_SPEC, _W_SPEC],
        out_specs=_X_SPEC, out_shape=_OUT_SHAPE,
    )(x, cnts, w)


def _tc_mid(z, cnts, b, w):
    return pl.pallas_call(
        _mid_body, grid=(N // RBLK,),
        in_specs=[_Z_SPEC, _Z_SPEC, _B_SPEC, _W_SPEC],
        out_specs=_X_SPEC, out_shape=_OUT_SHAPE,
    )(z, cnts, b, w)


def _tc_final(z, cnts, b):
    return pl.pallas_call(
        _final_body, grid=(N // RBLK,),
        in_specs=[_Z_SPEC, _Z_SPEC, _B_SPEC],
        out_specs=_X_SPEC, out_shape=_OUT_SHAPE,
    )(z, cnts, b)


def kernel(features, edge_index, W1, b1, W2, b2, W3, b3):
    src = edge_index[0]
    dst = edge_index[1]
    b1r = b1.reshape(1, D)
    b2r = b2.reshape(1, D)
    b3r = b3.reshape(1, D)
    cnts = _sc_degree_counts(src, dst)
    y1 = _tc_prep(features, cnts, W1)
    z1 = _sc_scatter(y1, src, dst)
    y2 = _tc_mid(z1, cnts, b1r, W2)
    z2 = _sc_scatter(y2, src, dst)
    y3 = _tc_mid(z2, cnts, b2r, W3)
    z3 = _sc_scatter(y3, src, dst)
    return _tc_final(z3, cnts, b3r)


# speedup vs baseline: 8.4885x; 1.1279x over previous
import functools

import jax
import jax.numpy as jnp
from jax import lax
from jax.experimental import pallas as pl
from jax.experimental.pallas import tpu as pltpu
from jax.experimental.pallas import tpu_sc as plsc

N = 10000
E = 320000
D = 128
NC = 2
NS = 16
NW = NC * NS
EPW = E // NW
CHUNK = 80
NCHUNKS = EPW // CHUNK
NBUF = 5
SBUF = 4
SFULL = NCHUNKS // SBUF
STAIL = NCHUNKS % SBUF
NRCHUNKS = N // CHUNK
CL = 16
RBLK = 1000
assert EPW % CHUNK == 0 and N % CHUNK == 0 and N % RBLK == 0
assert (EPW % 8 == 0) and (CHUNK % 8 == 0) and NCHUNKS % NBUF == 0


def _strided_row_chunks(s, body):
    def it(i, _):
        j = s + i * NS

        @pl.when(j < NRCHUNKS)
        def _():
            body(j * CHUNK)
        return 0

    lax.fori_loop(0, (NRCHUNKS + NS - 1) // NS, it, 0)


def _mesh():
    return plsc.VectorSubcoreMesh(core_axis_name="c", subcore_axis_name="s")


CL = 16


def _count_body(src_hbm, dst_hbm, out_s, out_d, acc_s, acc_d, buf,
                idx_s, idx_d, isem, asem):
    c = lax.axis_index("c")
    s = lax.axis_index("s")
    wid = c * NS + s

    def fill(val):
        def body(i, _):
            buf[i] = jnp.full((CL,), val, jnp.float32)
            return 0
        lax.fori_loop(0, CHUNK, body, 0)

    fill(0.0)

    def zero_chunk(rbase):
        pltpu.sync_copy(buf, acc_s.at[pl.ds(rbase, CHUNK)])
        pltpu.sync_copy(buf, acc_d.at[pl.ds(rbase, CHUNK)])

    _strided_row_chunks(s, zero_chunk)
    fill(1.0)
    plsc.subcore_barrier()

    def group_body(g, _):
        j0 = g * NBUF
        il, al = [], []
        for b in range(NBUF):
            ebase = wid * EPW + (j0 + b) * CHUNK
            il.append((
                pltpu.async_copy(src_hbm.at[pl.ds(ebase, CHUNK)],
                                 idx_s.at[b], isem.at[b]),
                pltpu.async_copy(dst_hbm.at[pl.ds(ebase, CHUNK)],
                                 idx_d.at[b], isem.at[b]),
            ))
        for b in range(NBUF):
            il[b][0].wait()
            il[b][1].wait()
            al.append((
                pltpu.async_copy(buf, acc_s.at[idx_s.at[b]], asem.at[b],
                                 add=True),
                pltpu.async_copy(buf, acc_d.at[idx_d.at[b]], asem.at[b],
                                 add=True),
            ))
        for b in range(NBUF):
            al[b][0].wait()
            al[b][1].wait()
        return 0

    lax.fori_loop(0, NCHUNKS // NBUF, group_body, 0)
    plsc.subcore_barrier()

    def flush_chunk(rbase):
        pltpu.sync_copy(acc_s.at[pl.ds(rbase, CHUNK)],
                        out_s.at[c, pl.ds(rbase, CHUNK)])
        pltpu.sync_copy(acc_d.at[pl.ds(rbase, CHUNK)],
                        out_d.at[c, pl.ds(rbase, CHUNK)])

    _strided_row_chunks(s, flush_chunk)


def _sc_degree_counts(src, dst):
    kern = pl.kernel(
        _count_body,
        out_type=(
            jax.ShapeDtypeStruct((NC, N, CL), jnp.float32),
            jax.ShapeDtypeStruct((NC, N, CL), jnp.float32),
        ),
        mesh=_mesh(),
        compiler_params=pltpu.CompilerParams(use_tc_tiling_on_sc=False),
        scratch_types=[
            pltpu.VMEM_SHARED((N, CL), jnp.float32),
            pltpu.VMEM_SHARED((N, CL), jnp.float32),
            pltpu.VMEM((CHUNK, CL), jnp.float32),
            pltpu.VMEM((NBUF, CHUNK), jnp.int32),
            pltpu.VMEM((NBUF, CHUNK), jnp.int32),
            pltpu.SemaphoreType.DMA((NBUF,)),
            pltpu.SemaphoreType.DMA((NBUF,)),
        ],
    )
    return kern(src, dst)


def _scatter_body(y_hbm, src_hbm, dst_hbm, out_hbm,
                  acc, idx_s, idx_d, rows, isem, gsem, ssem):
    c = lax.axis_index("c")
    s = lax.axis_index("s")
    wid = c * NS + s

    def zrow(i, _):
        for j in range(D // 16):
            rows[0, i, pl.ds(j * 16, 16)] = jnp.zeros((16,), jnp.float32)
        return 0

    lax.fori_loop(0, CHUNK, zrow, 0)
    _strided_row_chunks(s, lambda rbase: pltpu.sync_copy(
        rows.at[0], acc.at[pl.ds(rbase, CHUNK)]))
    plsc.subcore_barrier()

    def group_body(g, _):
        j0 = g * SBUF
        il, gl, sl = [], [], []
        for b in range(SBUF):
            ebase = wid * EPW + (j0 + b) * CHUNK
            il.append((
                pltpu.async_copy(src_hbm.at[pl.ds(ebase, CHUNK)],
                                 idx_s.at[b], isem.at[b]),
                pltpu.async_copy(dst_hbm.at[pl.ds(ebase, CHUNK)],
                                 idx_d.at[b], isem.at[b]),
            ))
        for b in range(SBUF):
            il[b][0].wait()
            gl.append(pltpu.async_copy(y_hbm.at[idx_s.at[b]], rows.at[b],
                                       gsem.at[b]))
        for b in range(SBUF):
            gl[b].wait()
            il[b][1].wait()
            sl.append(pltpu.async_copy(rows.at[b], acc.at[idx_d.at[b]],
                                       ssem.at[b], add=True))
        for b in range(SBUF):
            sl[b].wait()
        return 0

    lax.fori_loop(0, SFULL, group_body, 0)
    for t in range(STAIL):
        ebase = wid * EPW + (SFULL * SBUF + t) * CHUNK
        pltpu.sync_copy(src_hbm.at[pl.ds(ebase, CHUNK)], idx_s.at[0])
        pltpu.sync_copy(dst_hbm.at[pl.ds(ebase, CHUNK)], idx_d.at[0])
        pltpu.async_copy(y_hbm.at[idx_s.at[0]], rows.at[0], gsem.at[0]).wait()
        pltpu.sync_copy(rows.at[0], acc.at[idx_d.at[0]], add=True)
    plsc.subcore_barrier()
    _strided_row_chunks(s, lambda rbase: pltpu.sync_copy(
        acc.at[pl.ds(rbase, CHUNK)], out_hbm.at[c, pl.ds(rbase, CHUNK)]))


def _sc_scatter(y, src, dst):
    kern = pl.kernel(
        _scatter_body,
        out_type=jax.ShapeDtypeStruct((NC, N, D), jnp.float32),
        mesh=_mesh(),
        scratch_types=[
            pltpu.VMEM_SHARED((N, D), jnp.float32),
            pltpu.VMEM((SBUF, CHUNK), jnp.int32),
            pltpu.VMEM((SBUF, CHUNK), jnp.int32),
            pltpu.VMEM((SBUF, CHUNK, D), jnp.float32),
            pltpu.SemaphoreType.DMA((SBUF,)),
            pltpu.SemaphoreType.DMA((SBUF,)),
            pltpu.SemaphoreType.DMA((SBUF,)),
        ],
    )
    return kern(y, src, dst)


def _norm_from_counts(cnt):
    deg = cnt[0, :, 0:1] + cnt[1, :, 0:1]
    return jnp.where(deg > 0, lax.rsqrt(jnp.maximum(deg, 1.0)), 0.0)


def _dot(x, w):
    return jnp.dot(x, w, preferred_element_type=jnp.float32,
                   precision=lax.Precision.HIGHEST)


def _prep_body(x_ref, cs_ref, w_ref, o_ref):
    ns = _norm_from_counts(cs_ref[...])
    o_ref[...] = _dot(x_ref[...] * ns, w_ref[...])


def _mid_body(z_ref, cd_ref, cs_ref, b_ref, w_ref, o_ref):
    nd = _norm_from_counts(cd_ref[...])
    ns = _norm_from_counts(cs_ref[...])
    x = jnp.maximum(nd * (z_ref[0] + z_ref[1]) + b_ref[...], 0.0)
    o_ref[...] = _dot(x * ns, w_ref[...])


def _final_body(z_ref, cd_ref, b_ref, o_ref):
    nd = _norm_from_counts(cd_ref[...])
    v = nd * (z_ref[0] + z_ref[1]) + b_ref[...]
    t = v - jnp.max(v, axis=1, keepdims=True)
    o_ref[...] = t - jnp.log(jnp.sum(jnp.exp(t), axis=1, keepdims=True))


_Z_SPEC = pl.BlockSpec((NC, RBLK, D), lambda i: (0, i, 0))
_C_SPEC = pl.BlockSpec((NC, RBLK, CL), lambda i: (0, i, 0))
_B_SPEC = pl.BlockSpec((1, D), lambda i: (0, 0))
_W_SPEC = pl.BlockSpec((D, D), lambda i: (0, 0))
_X_SPEC = pl.BlockSpec((RBLK, D), lambda i: (i, 0))
_OUT_SHAPE = jax.ShapeDtypeStruct((N, D), jnp.float32)


def _tc_prep(x, cs, w):
    return pl.pallas_call(
        _prep_body, grid=(N // RBLK,),
        in_specs=[_X_SPEC, _C_SPEC, _W_SPEC],
        out_specs=_X_SPEC, out_shape=_OUT_SHAPE,
    )(x, cs, w)


def _tc_mid(z, cd, cs, b, w):
    return pl.pallas_call(
        _mid_body, grid=(N // RBLK,),
        in_specs=[_Z_SPEC, _C_SPEC, _C_SPEC, _B_SPEC, _W_SPEC],
        out_specs=_X_SPEC, out_shape=_OUT_SHAPE,
    )(z, cd, cs, b, w)


def _tc_final(z, cd, b):
    return pl.pallas_call(
        _final_body, grid=(N // RBLK,),
        in_specs=[_Z_SPEC, _C_SPEC, _B_SPEC],
        out_specs=_X_SPEC, out_shape=_OUT_SHAPE,
    )(z, cd, b)


def kernel(features, edge_index, W1, b1, W2, b2, W3, b3):
    src = edge_index[0]
    dst = edge_index[1]
    b1r = b1.reshape(1, D)
    b2r = b2.reshape(1, D)
    b3r = b3.reshape(1, D)
    cs, cd = _sc_degree_counts(src, dst)
    y1 = _tc_prep(features, cs, W1)
    z1 = _sc_scatter(y1, src, dst)
    y2 = _tc_mid(z1, cd, cs, b1r, W2)
    z2 = _sc_scatter(y2, src, dst)
    y3 = _tc_mid(z2, cd, cs, b2r, W3)
    z3 = _sc_scatter(y3, src, dst)
    return _tc_final(z3, cd, b3r)


# speedup vs baseline: 10.6871x; 1.2590x over previous
import functools

import jax
import jax.numpy as jnp
from jax import lax
from jax.experimental import pallas as pl
from jax.experimental.pallas import tpu as pltpu
from jax.experimental.pallas import tpu_sc as plsc

N = 10000
E = 320000
D = 128
NC = 2
NS = 16
NW = NC * NS
EPW = E // NW
CHUNK = 80
NCHUNKS = EPW // CHUNK
NBUF = 5
SBUF = 4
SFULL = NCHUNKS // SBUF
STAIL = NCHUNKS % SBUF
NRCHUNKS = N // CHUNK
CL = 16
RBLK = 1000
assert EPW % CHUNK == 0 and N % CHUNK == 0 and N % RBLK == 0
assert (EPW % 8 == 0) and (CHUNK % 8 == 0) and NCHUNKS % NBUF == 0


def _strided_row_chunks(s, body):
    def it(i, _):
        j = s + i * NS

        @pl.when(j < NRCHUNKS)
        def _():
            body(j * CHUNK)
        return 0

    lax.fori_loop(0, (NRCHUNKS + NS - 1) // NS, it, 0)


def _mesh():
    return plsc.VectorSubcoreMesh(core_axis_name="c", subcore_axis_name="s")


CL = 16


def _count_body(src_hbm, dst_hbm, out_s, out_d, acc_s, acc_d, buf,
                idx_s, idx_d, isem, asem):
    c = lax.axis_index("c")
    s = lax.axis_index("s")
    wid = c * NS + s

    def fill(val):
        def body(i, _):
            buf[i] = jnp.full((CL,), val, jnp.float32)
            return 0
        lax.fori_loop(0, CHUNK, body, 0)

    fill(0.0)

    def zero_chunk(rbase):
        pltpu.sync_copy(buf, acc_s.at[pl.ds(rbase, CHUNK)])
        pltpu.sync_copy(buf, acc_d.at[pl.ds(rbase, CHUNK)])

    _strided_row_chunks(s, zero_chunk)
    fill(1.0)
    plsc.subcore_barrier()

    def group_body(g, _):
        j0 = g * NBUF
        il, al = [], []
        for b in range(NBUF):
            ebase = wid * EPW + (j0 + b) * CHUNK
            il.append((
                pltpu.async_copy(src_hbm.at[pl.ds(ebase, CHUNK)],
                                 idx_s.at[b], isem.at[b]),
                pltpu.async_copy(dst_hbm.at[pl.ds(ebase, CHUNK)],
                                 idx_d.at[b], isem.at[b]),
            ))
        for b in range(NBUF):
            il[b][0].wait()
            il[b][1].wait()
            al.append((
                pltpu.async_copy(buf, acc_s.at[idx_s.at[b]], asem.at[b],
                                 add=True),
                pltpu.async_copy(buf, acc_d.at[idx_d.at[b]], asem.at[b],
                                 add=True),
            ))
        for b in range(NBUF):
            al[b][0].wait()
            al[b][1].wait()
        return 0

    lax.fori_loop(0, NCHUNKS // NBUF, group_body, 0)
    plsc.subcore_barrier()

    def flush_chunk(rbase):
        pltpu.sync_copy(acc_s.at[pl.ds(rbase, CHUNK)],
                        out_s.at[c, pl.ds(rbase, CHUNK)])
        pltpu.sync_copy(acc_d.at[pl.ds(rbase, CHUNK)],
                        out_d.at[c, pl.ds(rbase, CHUNK)])

    _strided_row_chunks(s, flush_chunk)


def _sc_degree_counts(src, dst):
    kern = pl.kernel(
        _count_body,
        out_type=(
            jax.ShapeDtypeStruct((NC, N, CL), jnp.float32),
            jax.ShapeDtypeStruct((NC, N, CL), jnp.float32),
        ),
        mesh=_mesh(),
        compiler_params=pltpu.CompilerParams(use_tc_tiling_on_sc=False),
        scratch_types=[
            pltpu.VMEM_SHARED((N, CL), jnp.float32),
            pltpu.VMEM_SHARED((N, CL), jnp.float32),
            pltpu.VMEM((CHUNK, CL), jnp.float32),
            pltpu.VMEM((NBUF, CHUNK), jnp.int32),
            pltpu.VMEM((NBUF, CHUNK), jnp.int32),
            pltpu.SemaphoreType.DMA((NBUF,)),
            pltpu.SemaphoreType.DMA((NBUF,)),
        ],
    )
    return kern(src, dst)


def _scatter_body(y_hbm, src_hbm, dst_hbm, out_hbm,
                  acc, idx_s, idx_d, rows, isem, gsem, ssem):
    c = lax.axis_index("c")
    s = lax.axis_index("s")
    wid = c * NS + s

    def zrow(i, _):
        for j in range(D // 16):
            rows[0, i, pl.ds(j * 16, 16)] = jnp.zeros((16,), jnp.float32)
        return 0

    lax.fori_loop(0, CHUNK, zrow, 0)
    _strided_row_chunks(s, lambda rbase: pltpu.sync_copy(
        rows.at[0], acc.at[pl.ds(rbase, CHUNK)]))
    plsc.subcore_barrier()

    def fire_idx(g, p, b):
        ebase = wid * EPW + (g * SBUF + b) * CHUNK
        ebase = jnp.minimum(ebase, E - CHUNK)
        pltpu.async_copy(src_hbm.at[pl.ds(ebase, CHUNK)],
                         idx_s.at[p, b], isem.at[p, b])
        pltpu.async_copy(dst_hbm.at[pl.ds(ebase, CHUNK)],
                         idx_d.at[p, b], isem.at[p, b])

    def wait_idx(p, b):
        pltpu.make_async_copy(src_hbm.at[pl.ds(0, CHUNK)],
                              idx_s.at[p, b], isem.at[p, b]).wait()
        pltpu.make_async_copy(dst_hbm.at[pl.ds(0, CHUNK)],
                              idx_d.at[p, b], isem.at[p, b]).wait()

    def fire_gather(p, b):
        pltpu.async_copy(y_hbm.at[idx_s.at[p, b]], rows.at[b], gsem.at[b])

    def wait_gather(b):
        pltpu.make_async_copy(y_hbm.at[idx_s.at[0, b]], rows.at[b],
                              gsem.at[b]).wait()

    def fire_scatter(p, b):
        pltpu.async_copy(rows.at[b], acc.at[idx_d.at[p, b]], ssem.at[b],
                         add=True)

    def wait_scatter(b):
        pltpu.make_async_copy(rows.at[b], acc.at[idx_d.at[0, b]],
                              ssem.at[b]).wait()

    def run_group(g, p, first):
        for b in range(SBUF):
            if not first:
                wait_scatter(b)
            wait_idx(p, b)
            fire_gather(p, b)
            fire_idx(g + 1, p ^ 1, b)
        for b in range(SBUF):
            wait_gather(b)
            fire_scatter(p, b)

    for b in range(SBUF):
        fire_idx(0, 0, b)
    run_group(0, 0, True)

    def pair_body(i, _):
        run_group(2 * i + 1, 1, False)
        run_group(2 * i + 2, 0, False)
        return 0

    lax.fori_loop(0, (SFULL - 1) // 2, pair_body, 0)
    for b in range(SBUF):
        wait_scatter(b)
    for t in range(STAIL):
        wait_idx(1, t)
        pltpu.async_copy(y_hbm.at[idx_s.at[1, t]], rows.at[t],
                         gsem.at[t]).wait()
        pltpu.sync_copy(rows.at[t], acc.at[idx_d.at[1, t]], add=True)
    plsc.subcore_barrier()
    _strided_row_chunks(s, lambda rbase: pltpu.sync_copy(
        acc.at[pl.ds(rbase, CHUNK)], out_hbm.at[c, pl.ds(rbase, CHUNK)]))


def _sc_scatter(y, src, dst):
    kern = pl.kernel(
        _scatter_body,
        out_type=jax.ShapeDtypeStruct((NC, N, D), jnp.float32),
        mesh=_mesh(),
        scratch_types=[
            pltpu.VMEM_SHARED((N, D), jnp.float32),
            pltpu.VMEM((2, SBUF, CHUNK), jnp.int32),
            pltpu.VMEM((2, SBUF, CHUNK), jnp.int32),
            pltpu.VMEM((SBUF, CHUNK, D), jnp.float32),
            pltpu.SemaphoreType.DMA((2, SBUF)),
            pltpu.SemaphoreType.DMA((SBUF,)),
            pltpu.SemaphoreType.DMA((SBUF,)),
        ],
    )
    return kern(y, src, dst)


def _norm_from_counts(cnt):
    deg = cnt[0, :, 0:1] + cnt[1, :, 0:1]
    return jnp.where(deg > 0, lax.rsqrt(jnp.maximum(deg, 1.0)), 0.0)


def _dot(x, w):
    return jnp.dot(x, w, preferred_element_type=jnp.float32,
                   precision=lax.Precision.HIGHEST)


def _prep_body(x_ref, cs_ref, w_ref, o_ref):
    ns = _norm_from_counts(cs_ref[...])
    o_ref[...] = _dot(x_ref[...] * ns, w_ref[...])


def _mid_body(z_ref, cd_ref, cs_ref, b_ref, w_ref, o_ref):
    nd = _norm_from_counts(cd_ref[...])
    ns = _norm_from_counts(cs_ref[...])
    x = jnp.maximum(nd * (z_ref[0] + z_ref[1]) + b_ref[...], 0.0)
    o_ref[...] = _dot(x * ns, w_ref[...])


def _final_body(z_ref, cd_ref, b_ref, o_ref):
    nd = _norm_from_counts(cd_ref[...])
    v = nd * (z_ref[0] + z_ref[1]) + b_ref[...]
    t = v - jnp.max(v, axis=1, keepdims=True)
    o_ref[...] = t - jnp.log(jnp.sum(jnp.exp(t), axis=1, keepdims=True))


_Z_SPEC = pl.BlockSpec((NC, RBLK, D), lambda i: (0, i, 0))
_C_SPEC = pl.BlockSpec((NC, RBLK, CL), lambda i: (0, i, 0))
_B_SPEC = pl.BlockSpec((1, D), lambda i: (0, 0))
_W_SPEC = pl.BlockSpec((D, D), lambda i: (0, 0))
_X_SPEC = pl.BlockSpec((RBLK, D), lambda i: (i, 0))
_OUT_SHAPE = jax.ShapeDtypeStruct((N, D), jnp.float32)


def _tc_prep(x, cs, w):
    return pl.pallas_call(
        _prep_body, grid=(N // RBLK,),
        in_specs=[_X_SPEC, _C_SPEC, _W_SPEC],
        out_specs=_X_SPEC, out_shape=_OUT_SHAPE,
    )(x, cs, w)


def _tc_mid(z, cd, cs, b, w):
    return pl.pallas_call(
        _mid_body, grid=(N // RBLK,),
        in_specs=[_Z_SPEC, _C_SPEC, _C_SPEC, _B_SPEC, _W_SPEC],
        out_specs=_X_SPEC, out_shape=_OUT_SHAPE,
    )(z, cd, cs, b, w)


def _tc_final(z, cd, b):
    return pl.pallas_call(
        _final_body, grid=(N // RBLK,),
        in_specs=[_Z_SPEC, _C_SPEC, _B_SPEC],
        out_specs=_X_SPEC, out_shape=_OUT_SHAPE,
    )(z, cd, b)


def kernel(features, edge_index, W1, b1, W2, b2, W3, b3):
    src = edge_index[0]
    dst = edge_index[1]
    b1r = b1.reshape(1, D)
    b2r = b2.reshape(1, D)
    b3r = b3.reshape(1, D)
    cs, cd = _sc_degree_counts(src, dst)
    y1 = _tc_prep(features, cs, W1)
    z1 = _sc_scatter(y1, src, dst)
    y2 = _tc_mid(z1, cd, cs, b1r, W2)
    z2 = _sc_scatter(y2, src, dst)
    y3 = _tc_mid(z2, cd, cs, b2r, W3)
    z3 = _sc_scatter(y3, src, dst)
    return _tc_final(z3, cd, b3r)


# speedup vs baseline: 10.7650x; 1.0073x over previous
import functools

import jax
import jax.numpy as jnp
from jax import lax
from jax.experimental import pallas as pl
from jax.experimental.pallas import tpu as pltpu
from jax.experimental.pallas import tpu_sc as plsc

N = 10000
E = 320000
D = 128
NC = 2
NS = 16
NW = NC * NS
EPW = E // NW
CHUNK = 80
NCHUNKS = EPW // CHUNK
NBUF = 5
SBUF = 4
SFULL = NCHUNKS // SBUF
STAIL = NCHUNKS % SBUF
NRCHUNKS = N // CHUNK
CL = 16
RBLK = 1000
assert EPW % CHUNK == 0 and N % CHUNK == 0 and N % RBLK == 0
assert (EPW % 8 == 0) and (CHUNK % 8 == 0) and NCHUNKS % NBUF == 0


def _strided_row_chunks(s, body):
    def it(i, _):
        j = s + i * NS

        @pl.when(j < NRCHUNKS)
        def _():
            body(j * CHUNK)
        return 0

    lax.fori_loop(0, (NRCHUNKS + NS - 1) // NS, it, 0)


def _mesh():
    return plsc.VectorSubcoreMesh(core_axis_name="c", subcore_axis_name="s")


CL = 16


def _count_body(src_hbm, dst_hbm, out_s, out_d, acc_s, acc_d, buf,
                idx_s, idx_d, isem, asem):
    c = lax.axis_index("c")
    s = lax.axis_index("s")
    wid = c * NS + s

    def fill(val):
        def body(i, _):
            buf[i] = jnp.full((CL,), val, jnp.float32)
            return 0
        lax.fori_loop(0, CHUNK, body, 0)

    fill(0.0)

    def zero_chunk(rbase):
        pltpu.sync_copy(buf, acc_s.at[pl.ds(rbase, CHUNK)])
        pltpu.sync_copy(buf, acc_d.at[pl.ds(rbase, CHUNK)])

    _strided_row_chunks(s, zero_chunk)
    fill(1.0)
    plsc.subcore_barrier()

    def group_body(g, _):
        j0 = g * NBUF
        il, al = [], []
        for b in range(NBUF):
            ebase = wid * EPW + (j0 + b) * CHUNK
            il.append((
                pltpu.async_copy(src_hbm.at[pl.ds(ebase, CHUNK)],
                                 idx_s.at[b], isem.at[b]),
                pltpu.async_copy(dst_hbm.at[pl.ds(ebase, CHUNK)],
                                 idx_d.at[b], isem.at[b]),
            ))
        for b in range(NBUF):
            il[b][0].wait()
            il[b][1].wait()
            al.append((
                pltpu.async_copy(buf, acc_s.at[idx_s.at[b]], asem.at[b],
                                 add=True),
                pltpu.async_copy(buf, acc_d.at[idx_d.at[b]], asem.at[b],
                                 add=True),
            ))
        for b in range(NBUF):
            al[b][0].wait()
            al[b][1].wait()
        return 0

    lax.fori_loop(0, NCHUNKS // NBUF, group_body, 0)
    plsc.subcore_barrier()

    def flush_chunk(rbase):
        pltpu.sync_copy(acc_s.at[pl.ds(rbase, CHUNK)],
                        out_s.at[c, pl.ds(rbase, CHUNK)])
        pltpu.sync_copy(acc_d.at[pl.ds(rbase, CHUNK)],
                        out_d.at[c, pl.ds(rbase, CHUNK)])

    _strided_row_chunks(s, flush_chunk)


def _sc_degree_counts(src, dst):
    kern = pl.kernel(
        _count_body,
        out_type=(
            jax.ShapeDtypeStruct((NC, N, CL), jnp.float32),
            jax.ShapeDtypeStruct((NC, N, CL), jnp.float32),
        ),
        mesh=_mesh(),
        compiler_params=pltpu.CompilerParams(use_tc_tiling_on_sc=False),
        scratch_types=[
            pltpu.VMEM_SHARED((N, CL), jnp.float32),
            pltpu.VMEM_SHARED((N, CL), jnp.float32),
            pltpu.VMEM((CHUNK, CL), jnp.float32),
            pltpu.VMEM((NBUF, CHUNK), jnp.int32),
            pltpu.VMEM((NBUF, CHUNK), jnp.int32),
            pltpu.SemaphoreType.DMA((NBUF,)),
            pltpu.SemaphoreType.DMA((NBUF,)),
        ],
    )
    return kern(src, dst)


def _scatter_body(y_hbm, src_hbm, dst_hbm, out_hbm,
                  acc, idx_s, idx_d, rows, isem, gsem, ssem):
    c = lax.axis_index("c")
    s = lax.axis_index("s")
    wid = c * NS + s

    def zrow(i, _):
        for j in range(D // 16):
            rows[0, i, pl.ds(j * 16, 16)] = jnp.zeros((16,), jnp.float32)
        return 0

    lax.fori_loop(0, CHUNK, zrow, 0)
    _strided_row_chunks(s, lambda rbase: pltpu.sync_copy(
        rows.at[0], acc.at[pl.ds(rbase, CHUNK)]))
    plsc.subcore_barrier()

    def fire_idx(g, p, b):
        ebase = wid * EPW + (g * SBUF + b) * CHUNK
        ebase = jnp.minimum(ebase, E - CHUNK)
        pltpu.async_copy(src_hbm.at[pl.ds(ebase, CHUNK)],
                         idx_s.at[p, b], isem.at[p, b])
        pltpu.async_copy(dst_hbm.at[pl.ds(ebase, CHUNK)],
                         idx_d.at[p, b], isem.at[p, b])

    def wait_idx(p, b):
        pltpu.make_async_copy(src_hbm.at[pl.ds(0, CHUNK)],
                              idx_s.at[p, b], isem.at[p, b]).wait()
        pltpu.make_async_copy(dst_hbm.at[pl.ds(0, CHUNK)],
                              idx_d.at[p, b], isem.at[p, b]).wait()

    def fire_gather(p, b):
        pltpu.async_copy(y_hbm.at[idx_s.at[p, b]], rows.at[b], gsem.at[b])

    def wait_gather(b):
        pltpu.make_async_copy(y_hbm.at[idx_s.at[0, b]], rows.at[b],
                              gsem.at[b]).wait()

    def fire_scatter(p, b):
        pltpu.async_copy(rows.at[b], acc.at[idx_d.at[p, b]], ssem.at[b],
                         add=True)

    def wait_scatter(b):
        pltpu.make_async_copy(rows.at[b], acc.at[idx_d.at[0, b]],
                              ssem.at[b]).wait()

    def run_group(g, p, first):
        for b in range(SBUF):
            if not first:
                wait_scatter(b)
            wait_idx(p, b)
            fire_gather(p, b)
            fire_idx(g + 1, p ^ 1, b)
        for b in range(SBUF):
            wait_gather(b)
            fire_scatter(p, b)

    for b in range(SBUF):
        fire_idx(0, 0, b)
    run_group(0, 0, True)

    def pair_body(i, _):
        run_group(2 * i + 1, 1, False)
        run_group(2 * i + 2, 0, False)
        return 0

    lax.fori_loop(0, (SFULL - 1) // 2, pair_body, 0)
    for b in range(SBUF):
        wait_scatter(b)
    for t in range(STAIL):
        wait_idx(1, t)
        pltpu.async_copy(y_hbm.at[idx_s.at[1, t]], rows.at[t],
                         gsem.at[t]).wait()
        pltpu.sync_copy(rows.at[t], acc.at[idx_d.at[1, t]], add=True)
    plsc.subcore_barrier()
    _strided_row_chunks(s, lambda rbase: pltpu.sync_copy(
        acc.at[pl.ds(rbase, CHUNK)], out_hbm.at[c, pl.ds(rbase, CHUNK)]))


def _sc_scatter(y, src, dst):
    kern = pl.kernel(
        _scatter_body,
        out_type=jax.ShapeDtypeStruct((NC, N, D), jnp.float32),
        mesh=_mesh(),
        scratch_types=[
            pltpu.VMEM_SHARED((N, D), jnp.float32),
            pltpu.VMEM((2, SBUF, CHUNK), jnp.int32),
            pltpu.VMEM((2, SBUF, CHUNK), jnp.int32),
            pltpu.VMEM((SBUF, CHUNK, D), jnp.float32),
            pltpu.SemaphoreType.DMA((2, SBUF)),
            pltpu.SemaphoreType.DMA((SBUF,)),
            pltpu.SemaphoreType.DMA((SBUF,)),
        ],
    )
    return kern(y, src, dst)


def _norm_from_counts(cnt):
    deg = cnt[0, :, 0:1] + cnt[1, :, 0:1]
    return jnp.where(deg > 0, lax.rsqrt(jnp.maximum(deg, 1.0)), 0.0)


def _dot(x, w):
    return jnp.dot(x, w, preferred_element_type=jnp.float32,
                   precision=lax.Precision.HIGHEST)


def _matmul_body(x_ref, w_ref, o_ref):
    o_ref[...] = _dot(x_ref[...], w_ref[...])


def _scale_body(x_ref, cs_ref, o_ref):
    ns = _norm_from_counts(cs_ref[...])
    o_ref[...] = x_ref[...] * ns


def _mid_body(z_ref, cd_ref, cs_ref, b_ref, w_ref, o_ref):
    nd = _norm_from_counts(cd_ref[...])
    ns = _norm_from_counts(cs_ref[...])
    x = jnp.maximum(nd * (z_ref[0] + z_ref[1]) + b_ref[...], 0.0)
    o_ref[...] = _dot(x * ns, w_ref[...])


def _final_body(z_ref, cd_ref, b_ref, o_ref):
    nd = _norm_from_counts(cd_ref[...])
    v = nd * (z_ref[0] + z_ref[1]) + b_ref[...]
    t = v - jnp.max(v, axis=1, keepdims=True)
    o_ref[...] = t - jnp.log(jnp.sum(jnp.exp(t), axis=1, keepdims=True))


_Z_SPEC = pl.BlockSpec((NC, RBLK, D), lambda i: (0, i, 0))
_C_SPEC = pl.BlockSpec((NC, RBLK, CL), lambda i: (0, i, 0))
_B_SPEC = pl.BlockSpec((1, D), lambda i: (0, 0))
_W_SPEC = pl.BlockSpec((D, D), lambda i: (0, 0))
_X_SPEC = pl.BlockSpec((RBLK, D), lambda i: (i, 0))
_OUT_SHAPE = jax.ShapeDtypeStruct((N, D), jnp.float32)


def _tc_matmul(x, w):
    return pl.pallas_call(
        _matmul_body, grid=(N // RBLK,),
        in_specs=[_X_SPEC, _W_SPEC],
        out_specs=_X_SPEC, out_shape=_OUT_SHAPE,
    )(x, w)


def _tc_scale(x, cs):
    return pl.pallas_call(
        _scale_body, grid=(N // RBLK,),
        in_specs=[_X_SPEC, _C_SPEC],
        out_specs=_X_SPEC, out_shape=_OUT_SHAPE,
    )(x, cs)


def _tc_mid(z, cd, cs, b, w):
    return pl.pallas_call(
        _mid_body, grid=(N // RBLK,),
        in_specs=[_Z_SPEC, _C_SPEC, _C_SPEC, _B_SPEC, _W_SPEC],
        out_specs=_X_SPEC, out_shape=_OUT_SHAPE,
    )(z, cd, cs, b, w)


def _tc_final(z, cd, b):
    return pl.pallas_call(
        _final_body, grid=(N // RBLK,),
        in_specs=[_Z_SPEC, _C_SPEC, _B_SPEC],
        out_specs=_X_SPEC, out_shape=_OUT_SHAPE,
    )(z, cd, b)


def kernel(features, edge_index, W1, b1, W2, b2, W3, b3):
    src = edge_index[0]
    dst = edge_index[1]
    b1r = b1.reshape(1, D)
    b2r = b2.reshape(1, D)
    b3r = b3.reshape(1, D)
    u1 = _tc_matmul(features, W1)
    cs, cd = _sc_degree_counts(src, dst)
    y1 = _tc_scale(u1, cs)
    z1 = _sc_scatter(y1, src, dst)
    y2 = _tc_mid(z1, cd, cs, b1r, W2)
    z2 = _sc_scatter(y2, src, dst)
    y3 = _tc_mid(z2, cd, cs, b2r, W3)
    z3 = _sc_scatter(y3, src, dst)
    return _tc_final(z3, cd, b3r)
